# baseline (speedup 1.0000x reference)
import jax
import jax.numpy as jnp
from jax import lax
from jax.experimental import pallas as pl
from jax.experimental.pallas import tpu as pltpu
from jax.experimental.pallas import tpu_sc as plsc

B = 32
L = 4
V = 100000
N = B * L
PLACEHOLDER = -1

LANES = 16
VC = 128
SHARE = 3072
NMAIN = SHARE // VC
EXTRA0 = 32 * SHARE
MICRO0 = EXTRA0 + 13 * VC
MICRO_W = V - MICRO0
I32_MAX = 2147483647


def _sc_body(ids_hbm, d_hbm, t_hbm, q_hbm,
             scm_out, sci_out, scdp_out, sctp_out,
             ids_v,
             tA, dA, qA, tB, dB, qB, tM, dM, qM, rqf,
             mst, ist, dpst, tpst,
             gm, gi, gdp, gtp, obf, obi,
             stage_m, stage_i, stage_dp, stage_tp,
             semA, semB):
    c_ax = lax.axis_index("c")
    s_ax = lax.axis_index("s")
    w = c_ax * 16 + s_ax
    iota = lax.iota(jnp.int32, LANES)
    base = w * SHARE

    tsets = (tA, tB)
    dsets = (dA, dB)
    qsets = (qA, qB)
    sems = (semA, semB)

    pltpu.sync_copy(ids_hbm, ids_v)
    idvs = [plsc.load_gather(ids_v, [16 * k + iota]) for k in range(8)]
    qrows = [4 * k + jnp.right_shift(iota, 2) for k in range(8)]
    rowsk = [16 * k + iota for k in range(8)]

    def issue(n, s):
        voc0 = base + n * VC
        return [
            pltpu.async_copy(t_hbm.at[pl.ds(voc0 * 128, VC * 128)],
                             tsets[s], sems[s]),
            pltpu.async_copy(d_hbm.at[pl.ds(voc0 * 128, VC * 128)],
                             dsets[s], sems[s]),
            pltpu.async_copy(q_hbm.at[pl.ds(0, 32), pl.ds(voc0, VC)],
                             qsets[s], sems[s]),
        ]

    def build_rq(qbuf, width):
        nvec = (32 * width) // LANES
        per_row = width // LANES
        def body(i, _):
            row = i // per_row
            col = (i % per_row) * LANES
            rqf[pl.ds(i * LANES, LANES)] = 1.0 / qbuf[row, pl.ds(col, LANES)]
            return 0
        lax.fori_loop(0, nvec, body, 0, unroll=4)

    def make_inner(tbuf, dbuf, width, voc0, qbases):
        def inner(j, carry):
            ms = list(carry[:8])
            ix = list(carry[8:])
            jsplat = jnp.full((LANES,), j, jnp.int32)
            vidx = jnp.full((LANES,), voc0, jnp.int32) + j
            for k in range(8):
                qg = plsc.load_gather(rqf, [qbases[k] + jsplat])
                p = jnp.maximum(tbuf[pl.ds(j * 128 + 16 * k, 16)]
                                - dbuf[pl.ds(j * 128 + 16 * k, 16)], 0.0)
                r = p * qg
                better = r > ms[k]
                ms[k] = jnp.maximum(ms[k], r)
                ix[k] = jnp.where(better, vidx, ix[k])
            return tuple(ms) + tuple(ix)
        return inner

    def extract(tbuf, dbuf, voc0, width, dps, tps):
        ndp, ntp = [], []
        for k in range(8):
            in_r = (idvs[k] >= voc0) & (idvs[k] < voc0 + width)
            pos = jnp.clip(idvs[k] - voc0, 0, width - 1)
            flat = pos * 128 + rowsk[k]
            g_t = plsc.load_gather(tbuf, [flat])
            g_d = plsc.load_gather(dbuf, [flat])
            ndp.append(jnp.where(in_r, g_d, dps[k]))
            ntp.append(jnp.where(in_r, g_t, tps[k]))
        return ndp, ntp

    zero_f = jnp.zeros((LANES,), jnp.float32)
    m = [jnp.full((LANES,), -1.0, jnp.float32)] * 8
    mi = [jnp.zeros((LANES,), jnp.int32)] * 8
    dps = [zero_f] * 8
    tps = [zero_f] * 8
    qbases_vc = [qrows[k] * VC for k in range(8)]

    cur = issue(0, 0)
    for n in range(NMAIN):
        s = n % 2
        voc0 = base + n * VC
        for cp in cur:
            cp.wait()
        if n + 1 < NMAIN:
            cur = issue(n + 1, (n + 1) % 2)
        build_rq(qsets[s], VC)
        carry = lax.fori_loop(
            0, VC, make_inner(tsets[s], dsets[s], VC, voc0, qbases_vc),
            tuple(m) + tuple(mi), unroll=4)
        m = list(carry[:8])
        mi = list(carry[8:])
        dps, tps = extract(tsets[s], dsets[s], voc0, VC, dps, tps)

    for k in range(8):
        mst[pl.ds(16 * k, 16)] = m[k]
        ist[pl.ds(16 * k, 16)] = mi[k]
        dpst[pl.ds(16 * k, 16)] = dps[k]
        tpst[pl.ds(16 * k, 16)] = tps[k]

    def ref_chunk(voc0, width, tb, db, qb):
        pltpu.sync_copy(t_hbm.at[pl.ds(voc0 * 128, width * 128)], tb)
        pltpu.sync_copy(d_hbm.at[pl.ds(voc0 * 128, width * 128)], db)
        pltpu.sync_copy(q_hbm.at[pl.ds(0, 32), pl.ds(voc0, width)], qb)
        build_rq(qb, width)
        qb_w = [qrows[k] * width for k in range(8)]
        m_l = [mst[pl.ds(16 * k, 16)] for k in range(8)]
        i_l = [ist[pl.ds(16 * k, 16)] for k in range(8)]
        carry = lax.fori_loop(0, width, make_inner(tb, db, width, voc0, qb_w),
                              tuple(m_l) + tuple(i_l), unroll=4)
        d_l = [dpst[pl.ds(16 * k, 16)] for k in range(8)]
        t_l = [tpst[pl.ds(16 * k, 16)] for k in range(8)]
        nd, nt = extract(tb, db, voc0, width, d_l, t_l)
        for k in range(8):
            mst[pl.ds(16 * k, 16)] = carry[k]
            ist[pl.ds(16 * k, 16)] = carry[8 + k]
            dpst[pl.ds(16 * k, 16)] = nd[k]
            tpst[pl.ds(16 * k, 16)] = nt[k]

    has_extra = ((c_ax == 0) & (s_ax < 7)) | ((c_ax == 1) & (s_ax < 6))
    e_idx = jnp.where(c_ax == 0, s_ax, 7 + s_ax)

    @pl.when(has_extra)
    def _():
        ref_chunk(EXTRA0 + e_idx * VC, VC, tA, dA, qA)

    @pl.when((c_ax == 1) & (s_ax == 15))
    def _():
        ref_chunk(MICRO0, MICRO_W, tM, dM, qM)

    pltpu.sync_copy(mst, stage_m.at[pl.ds(s_ax * 128, 128)])
    pltpu.sync_copy(ist, stage_i.at[pl.ds(s_ax * 128, 128)])
    pltpu.sync_copy(dpst, stage_dp.at[pl.ds(s_ax * 128, 128)])
    pltpu.sync_copy(tpst, stage_tp.at[pl.ds(s_ax * 128, 128)])
    plsc.subcore_barrier()
    pltpu.sync_copy(stage_m, gm)
    pltpu.sync_copy(stage_i, gi)
    pltpu.sync_copy(stage_dp, gdp)
    pltpu.sync_copy(stage_tp, gtp)

    romax = zero_f
    roidx = jnp.zeros((LANES,), jnp.int32)
    rodp = zero_f
    rotp = zero_f
    for dr in range(8):
        r = 8 * s_ax + dr
        idxv = iota * 128 + r
        mg = plsc.load_gather(gm, [idxv])
        ig = plsc.load_gather(gi, [idxv])
        dg = plsc.load_gather(gdp, [idxv])
        tg = plsc.load_gather(gtp, [idxv])
        mv = jnp.max(mg)
        bi = jnp.min(jnp.where(mg == mv, ig, I32_MAX))
        sel = iota == dr
        romax = jnp.where(sel, mv, romax)
        roidx = jnp.where(sel, bi, roidx)
        rodp = jnp.where(sel, jnp.sum(dg), rodp)
        rotp = jnp.where(sel, jnp.sum(tg), rotp)

    off = c_ax * 128 + 8 * s_ax
    obf[...] = romax
    pltpu.sync_copy(obf.at[pl.ds(0, 8)], scm_out.at[pl.ds(off, 8)])
    obi[...] = roidx
    pltpu.sync_copy(obi.at[pl.ds(0, 8)], sci_out.at[pl.ds(off, 8)])
    obf[...] = rodp
    pltpu.sync_copy(obf.at[pl.ds(0, 8)], scdp_out.at[pl.ds(off, 8)])
    obf[...] = rotp
    pltpu.sync_copy(obf.at[pl.ds(0, 8)], sctp_out.at[pl.ds(off, 8)])


def _tc_body(scm, sci, scdp, sctp, ids2, u2, bon, out):
    m0 = scm[0]
    m1 = scm[1]
    mv = jnp.maximum(m0, m1)
    b0 = jnp.where(m0 == mv, sci[0], I32_MAX)
    b1 = jnp.where(m1 == mv, sci[1], I32_MAX)
    rec = jnp.minimum(b0, b1)
    dp = scdp[0] + scdp[1]
    tp = sctp[0] + sctp[1]
    safe_dp = jnp.where(dp > 0, dp, 1.0)
    acc = (dp > 0) & (tp / safe_dp >= u2[...])
    tok = jnp.where(acc, ids2[...], rec)
    a0 = acc[:, 0:1]
    a1 = acc[:, 1:2]
    a2 = acc[:, 2:3]
    w2 = a0 & a1
    w3 = w2 & a2
    allacc = w3 & acc[:, 3:4]
    neg = jnp.full((B, 1), PLACEHOLDER, jnp.int32)
    out[:, 0:1] = tok[:, 0:1]
    out[:, 1:2] = jnp.where(a0, tok[:, 1:2], neg)
    out[:, 2:3] = jnp.where(w2, tok[:, 2:3], neg)
    out[:, 3:4] = jnp.where(w3, tok[:, 3:4], neg)
    out[:, 4:5] = jnp.where(allacc, bon[...], neg)
    out[:, 5:8] = jnp.broadcast_to(neg, (B, 3))


@jax.jit
def _run(ids, dp2, tp2, bonus, u, q2):
    mesh = plsc.VectorSubcoreMesh(core_axis_name="c", subcore_axis_name="s")
    fsds = lambda n: jax.ShapeDtypeStruct((n,), jnp.float32)
    isds = lambda n: jax.ShapeDtypeStruct((n,), jnp.int32)
    sc = pl.kernel(
        _sc_body,
        out_type=(fsds(256), isds(256), fsds(256), fsds(256)),
        mesh=mesh,
        scratch_types=[
            pltpu.VMEM((N,), jnp.int32),
            pltpu.VMEM((VC * 128,), jnp.float32),
            pltpu.VMEM((VC * 128,), jnp.float32),
            pltpu.VMEM((32, VC), jnp.float32),
            pltpu.VMEM((VC * 128,), jnp.float32),
            pltpu.VMEM((VC * 128,), jnp.float32),
            pltpu.VMEM((32, VC), jnp.float32),
            pltpu.VMEM((MICRO_W * 128,), jnp.float32),
            pltpu.VMEM((MICRO_W * 128,), jnp.float32),
            pltpu.VMEM((32, MICRO_W), jnp.float32),
            pltpu.VMEM((32 * VC,), jnp.float32),
            pltpu.VMEM((128,), jnp.float32),
            pltpu.VMEM((128,), jnp.int32),
            pltpu.VMEM((128,), jnp.float32),
            pltpu.VMEM((128,), jnp.float32),
            pltpu.VMEM((2048,), jnp.float32),
            pltpu.VMEM((2048,), jnp.int32),
            pltpu.VMEM((2048,), jnp.float32),
            pltpu.VMEM((2048,), jnp.float32),
            pltpu.VMEM((LANES,), jnp.float32),
            pltpu.VMEM((LANES,), jnp.int32),
            pltpu.VMEM_SHARED((2048,), jnp.float32),
            pltpu.VMEM_SHARED((2048,), jnp.int32),
            pltpu.VMEM_SHARED((2048,), jnp.float32),
            pltpu.VMEM_SHARED((2048,), jnp.float32),
            pltpu.SemaphoreType.DMA, pltpu.SemaphoreType.DMA,
        ],
        compiler_params=pltpu.CompilerParams(needs_layout_passes=False),
    )
    scm, sci, scdp, sctp = sc(ids, dp2.T.reshape(-1), tp2.T.reshape(-1), q2)
    out = pl.pallas_call(
        _tc_body,
        out_shape=jax.ShapeDtypeStruct((B, 8), jnp.int32),
    )(scm.reshape(2, B, L), sci.reshape(2, B, L),
      scdp.reshape(2, B, L), sctp.reshape(2, B, L),
      ids.reshape(B, L), u.reshape(B, L), bonus.reshape(B, 1))
    return out


def kernel(draft_token_ids, cu_num_draft_tokens, draft_probs, target_probs,
           bonus_token_ids, uniform_probs, q):
    del cu_num_draft_tokens
    out = _run(draft_token_ids, draft_probs, target_probs,
               bonus_token_ids, uniform_probs, q)
    return out[:, :L + 1]


# speedup vs baseline: 1.9413x; 1.9413x over previous
import jax
import jax.numpy as jnp
from jax import lax
from jax.experimental import pallas as pl
from jax.experimental.pallas import tpu as pltpu
from jax.experimental.pallas import tpu_sc as plsc

B = 32
L = 4
V = 100000
N = B * L
PLACEHOLDER = -1

LANES = 16
VC = 128
SHARE = 3072
NMAIN = SHARE // VC
EXTRA0 = 32 * SHARE
MICRO0 = EXTRA0 + 13 * VC
MICRO_W = V - MICRO0
I32_MAX = 2147483647


def _sc_body(ids_hbm, d_hbm, t_hbm, q_hbm,
             scm_out, sci_out, scdp_out, sctp_out,
             ids_v,
             tA, dA, qA, tB, dB, qB, tM, dM, qM, rqexp,
             mst, ist, dpst, tpst,
             gm, gi, gdp, gtp, obf, obi,
             stage_m, stage_i, stage_dp, stage_tp,
             semA, semB):
    c_ax = lax.axis_index("c")
    s_ax = lax.axis_index("s")
    w = c_ax * 16 + s_ax
    iota = lax.iota(jnp.int32, LANES)
    base = w * SHARE

    tsets = (tA, tB)
    dsets = (dA, dB)
    qsets = (qA, qB)
    sems = (semA, semB)

    pltpu.sync_copy(ids_hbm, ids_v)
    idvs = [plsc.load_gather(ids_v, [16 * k + iota]) for k in range(8)]
    rowsk = [16 * k + iota for k in range(8)]

    def issue(n, s):
        voc0 = base + n * VC
        return [
            pltpu.async_copy(t_hbm.at[pl.ds(voc0 * 128, VC * 128)],
                             tsets[s], sems[s]),
            pltpu.async_copy(d_hbm.at[pl.ds(voc0 * 128, VC * 128)],
                             dsets[s], sems[s]),
            pltpu.async_copy(q_hbm.at[pl.ds(0, 32), pl.ds(voc0, VC)],
                             qsets[s], sems[s]),
        ]

    lane4 = 4 * iota

    def make_inner(tbuf, dbuf, qbuf, voc0):
        def inner(j, carry):
            ms = list(carry[:8])
            ix = list(carry[8:])
            jsplat = jnp.full((LANES,), j, jnp.int32)
            vidx = jnp.full((LANES,), voc0, jnp.int32) + j
            jb = jsplat * 128
            rq1 = 1.0 / plsc.load_gather(qbuf, [iota, jsplat])
            rq2 = 1.0 / plsc.load_gather(qbuf, [16 + iota, jsplat])
            for p in range(4):
                plsc.store_scatter(rqexp, [jb + (lane4 + p)], rq1)
                plsc.store_scatter(rqexp, [jb + (lane4 + (64 + p))], rq2)
            for k in range(8):
                qg = rqexp[pl.ds(j * 128 + 16 * k, 16)]
                p = jnp.maximum(tbuf[pl.ds(j * 128 + 16 * k, 16)]
                                - dbuf[pl.ds(j * 128 + 16 * k, 16)], 0.0)
                r = p * qg
                better = r > ms[k]
                ms[k] = jnp.maximum(ms[k], r)
                ix[k] = jnp.where(better, vidx, ix[k])
            return tuple(ms) + tuple(ix)
        return inner

    def extract(tbuf, dbuf, voc0, width, dps, tps):
        ndp, ntp = [], []
        for k in range(8):
            in_r = (idvs[k] >= voc0) & (idvs[k] < voc0 + width)
            pos = jnp.clip(idvs[k] - voc0, 0, width - 1)
            flat = pos * 128 + rowsk[k]
            g_t = plsc.load_gather(tbuf, [flat])
            g_d = plsc.load_gather(dbuf, [flat])
            ndp.append(jnp.where(in_r, g_d, dps[k]))
            ntp.append(jnp.where(in_r, g_t, tps[k]))
        return ndp, ntp

    zero_f = jnp.zeros((LANES,), jnp.float32)
    m = [jnp.full((LANES,), -1.0, jnp.float32)] * 8
    mi = [jnp.zeros((LANES,), jnp.int32)] * 8
    dps = [zero_f] * 8
    tps = [zero_f] * 8

    cur = issue(0, 0)
    for n in range(NMAIN):
        s = n % 2
        voc0 = base + n * VC
        for cp in cur:
            cp.wait()
        if n + 1 < NMAIN:
            cur = issue(n + 1, (n + 1) % 2)
        carry = lax.fori_loop(
            0, VC, make_inner(tsets[s], dsets[s], qsets[s], voc0),
            tuple(m) + tuple(mi), unroll=4)
        m = list(carry[:8])
        mi = list(carry[8:])
        dps, tps = extract(tsets[s], dsets[s], voc0, VC, dps, tps)

    for k in range(8):
        mst[pl.ds(16 * k, 16)] = m[k]
        ist[pl.ds(16 * k, 16)] = mi[k]
        dpst[pl.ds(16 * k, 16)] = dps[k]
        tpst[pl.ds(16 * k, 16)] = tps[k]

    def ref_chunk(voc0, width, tb, db, qb):
        pltpu.sync_copy(t_hbm.at[pl.ds(voc0 * 128, width * 128)], tb)
        pltpu.sync_copy(d_hbm.at[pl.ds(voc0 * 128, width * 128)], db)
        pltpu.sync_copy(q_hbm.at[pl.ds(0, 32), pl.ds(voc0, width)], qb)
        m_l = [mst[pl.ds(16 * k, 16)] for k in range(8)]
        i_l = [ist[pl.ds(16 * k, 16)] for k in range(8)]
        carry = lax.fori_loop(0, width, make_inner(tb, db, qb, voc0),
                              tuple(m_l) + tuple(i_l), unroll=4)
        d_l = [dpst[pl.ds(16 * k, 16)] for k in range(8)]
        t_l = [tpst[pl.ds(16 * k, 16)] for k in range(8)]
        nd, nt = extract(tb, db, voc0, width, d_l, t_l)
        for k in range(8):
            mst[pl.ds(16 * k, 16)] = carry[k]
            ist[pl.ds(16 * k, 16)] = carry[8 + k]
            dpst[pl.ds(16 * k, 16)] = nd[k]
            tpst[pl.ds(16 * k, 16)] = nt[k]

    has_extra = ((c_ax == 0) & (s_ax < 7)) | ((c_ax == 1) & (s_ax < 6))
    e_idx = jnp.where(c_ax == 0, s_ax, 7 + s_ax)

    @pl.when(has_extra)
    def _():
        ref_chunk(EXTRA0 + e_idx * VC, VC, tA, dA, qA)

    @pl.when((c_ax == 1) & (s_ax == 15))
    def _():
        ref_chunk(MICRO0, MICRO_W, tM, dM, qM)

    pltpu.sync_copy(mst, stage_m.at[pl.ds(s_ax * 128, 128)])
    pltpu.sync_copy(ist, stage_i.at[pl.ds(s_ax * 128, 128)])
    pltpu.sync_copy(dpst, stage_dp.at[pl.ds(s_ax * 128, 128)])
    pltpu.sync_copy(tpst, stage_tp.at[pl.ds(s_ax * 128, 128)])
    plsc.subcore_barrier()
    pltpu.sync_copy(stage_m, gm)
    pltpu.sync_copy(stage_i, gi)
    pltpu.sync_copy(stage_dp, gdp)
    pltpu.sync_copy(stage_tp, gtp)

    romax = zero_f
    roidx = jnp.zeros((LANES,), jnp.int32)
    rodp = zero_f
    rotp = zero_f
    for dr in range(8):
        r = 8 * s_ax + dr
        idxv = iota * 128 + r
        mg = plsc.load_gather(gm, [idxv])
        ig = plsc.load_gather(gi, [idxv])
        dg = plsc.load_gather(gdp, [idxv])
        tg = plsc.load_gather(gtp, [idxv])
        mv = jnp.max(mg)
        bi = jnp.min(jnp.where(mg == mv, ig, I32_MAX))
        sel = iota == dr
        romax = jnp.where(sel, mv, romax)
        roidx = jnp.where(sel, bi, roidx)
        rodp = jnp.where(sel, jnp.sum(dg), rodp)
        rotp = jnp.where(sel, jnp.sum(tg), rotp)

    off = c_ax * 128 + 8 * s_ax
    obf[...] = romax
    pltpu.sync_copy(obf.at[pl.ds(0, 8)], scm_out.at[pl.ds(off, 8)])
    obi[...] = roidx
    pltpu.sync_copy(obi.at[pl.ds(0, 8)], sci_out.at[pl.ds(off, 8)])
    obf[...] = rodp
    pltpu.sync_copy(obf.at[pl.ds(0, 8)], scdp_out.at[pl.ds(off, 8)])
    obf[...] = rotp
    pltpu.sync_copy(obf.at[pl.ds(0, 8)], sctp_out.at[pl.ds(off, 8)])


def _tc_body(scm, sci, scdp, sctp, ids2, u2, bon, out):
    m0 = scm[0]
    m1 = scm[1]
    mv = jnp.maximum(m0, m1)
    b0 = jnp.where(m0 == mv, sci[0], I32_MAX)
    b1 = jnp.where(m1 == mv, sci[1], I32_MAX)
    rec = jnp.minimum(b0, b1)
    dp = scdp[0] + scdp[1]
    tp = sctp[0] + sctp[1]
    safe_dp = jnp.where(dp > 0, dp, 1.0)
    acc = (dp > 0) & (tp / safe_dp >= u2[...])
    tok = jnp.where(acc, ids2[...], rec)
    a0 = acc[:, 0:1]
    a1 = acc[:, 1:2]
    a2 = acc[:, 2:3]
    w2 = a0 & a1
    w3 = w2 & a2
    allacc = w3 & acc[:, 3:4]
    neg = jnp.full((B, 1), PLACEHOLDER, jnp.int32)
    out[:, 0:1] = tok[:, 0:1]
    out[:, 1:2] = jnp.where(a0, tok[:, 1:2], neg)
    out[:, 2:3] = jnp.where(w2, tok[:, 2:3], neg)
    out[:, 3:4] = jnp.where(w3, tok[:, 3:4], neg)
    out[:, 4:5] = jnp.where(allacc, bon[...], neg)
    out[:, 5:8] = jnp.broadcast_to(neg, (B, 3))


@jax.jit
def _run(ids, dp2, tp2, bonus, u, q2):
    mesh = plsc.VectorSubcoreMesh(core_axis_name="c", subcore_axis_name="s")
    fsds = lambda n: jax.ShapeDtypeStruct((n,), jnp.float32)
    isds = lambda n: jax.ShapeDtypeStruct((n,), jnp.int32)
    sc = pl.kernel(
        _sc_body,
        out_type=(fsds(256), isds(256), fsds(256), fsds(256)),
        mesh=mesh,
        scratch_types=[
            pltpu.VMEM((N,), jnp.int32),
            pltpu.VMEM((VC * 128,), jnp.float32),
            pltpu.VMEM((VC * 128,), jnp.float32),
            pltpu.VMEM((32, VC), jnp.float32),
            pltpu.VMEM((VC * 128,), jnp.float32),
            pltpu.VMEM((VC * 128,), jnp.float32),
            pltpu.VMEM((32, VC), jnp.float32),
            pltpu.VMEM((MICRO_W * 128,), jnp.float32),
            pltpu.VMEM((MICRO_W * 128,), jnp.float32),
            pltpu.VMEM((32, MICRO_W), jnp.float32),
            pltpu.VMEM((VC * 128,), jnp.float32),
            pltpu.VMEM((128,), jnp.float32),
            pltpu.VMEM((128,), jnp.int32),
            pltpu.VMEM((128,), jnp.float32),
            pltpu.VMEM((128,), jnp.float32),
            pltpu.VMEM((2048,), jnp.float32),
            pltpu.VMEM((2048,), jnp.int32),
            pltpu.VMEM((2048,), jnp.float32),
            pltpu.VMEM((2048,), jnp.float32),
            pltpu.VMEM((LANES,), jnp.float32),
            pltpu.VMEM((LANES,), jnp.int32),
            pltpu.VMEM_SHARED((2048,), jnp.float32),
            pltpu.VMEM_SHARED((2048,), jnp.int32),
            pltpu.VMEM_SHARED((2048,), jnp.float32),
            pltpu.VMEM_SHARED((2048,), jnp.float32),
            pltpu.SemaphoreType.DMA, pltpu.SemaphoreType.DMA,
        ],
        compiler_params=pltpu.CompilerParams(needs_layout_passes=False),
    )
    scm, sci, scdp, sctp = sc(ids, dp2.T.reshape(-1), tp2.T.reshape(-1), q2)
    out = pl.pallas_call(
        _tc_body,
        out_shape=jax.ShapeDtypeStruct((B, 8), jnp.int32),
    )(scm.reshape(2, B, L), sci.reshape(2, B, L),
      scdp.reshape(2, B, L), sctp.reshape(2, B, L),
      ids.reshape(B, L), u.reshape(B, L), bonus.reshape(B, 1))
    return out


def kernel(draft_token_ids, cu_num_draft_tokens, draft_probs, target_probs,
           bonus_token_ids, uniform_probs, q):
    del cu_num_draft_tokens
    out = _run(draft_token_ids, draft_probs, target_probs,
               bonus_token_ids, uniform_probs, q)
    return out[:, :L + 1]
